# baseline (speedup 1.0000x reference)
import dataclasses
import functools

import jax
import jax.numpy as jnp
from jax import lax
from jax.experimental import pallas as pl
from jax.experimental.pallas import tpu as pltpu
from jax.experimental.pallas import tpu_sc as plsc

N_NODES = 10000
N_EDGES = 320000
D = 128
DW = D // 2

N_TILES = 32
EDGES_PAD = 327680
EDGES_PER_TILE = EDGES_PAD // N_TILES
CHUNK = 128
N_CHUNKS = EDGES_PER_TILE // CHUNK
IDX_ROWS = CHUNK // 128
IDX_ROWS_TILE = EDGES_PER_TILE // 128


def _normalize_body(x_ref, o_ref):
    xb = x_ref[...]
    ss = jnp.sum(xb * xb, axis=1, keepdims=True)
    norm = jnp.maximum(jnp.sqrt(ss), 1e-12)
    o_ref[...] = xb / norm


def _normalize(x):
    return pl.pallas_call(
        _normalize_body,
        out_shape=jax.ShapeDtypeStruct((N_NODES, D), jnp.float32),
        grid=(10,),
        in_specs=[pl.BlockSpec((N_NODES // 10, D), lambda i: (i, 0))],
        out_specs=pl.BlockSpec((N_NODES // 10, D), lambda i: (i, 0)),
    )(x)


def _sc_cosine(norm_h, src2d, dst2d):
    mesh = plsc.VectorSubcoreMesh(core_axis_name="c", subcore_axis_name="s")
    cp = pltpu.CompilerParams()
    if "needs_layout_passes" in pltpu.CompilerParams.__dataclass_fields__:
        cp = dataclasses.replace(cp, needs_layout_passes=False)

    @functools.partial(
        pl.kernel,
        mesh=mesh,
        compiler_params=cp,
        out_type=jax.ShapeDtypeStruct((EDGES_PAD,), jnp.float32),
        scratch_types=[
            pltpu.VMEM((IDX_ROWS_TILE, 128), jnp.int32),
            pltpu.VMEM((IDX_ROWS_TILE, 128), jnp.int32),
            pltpu.VMEM((CHUNK, D), jnp.float32),
            pltpu.VMEM((CHUNK, D), jnp.float32),
            pltpu.VMEM((CHUNK, D), jnp.float32),
            pltpu.VMEM((CHUNK, D), jnp.float32),
            pltpu.VMEM((EDGES_PER_TILE,), jnp.float32),
            pltpu.SemaphoreType.DMA,
            pltpu.SemaphoreType.DMA,
        ],
    )
    def sc_kernel(h_hbm, src_hbm, dst_hbm, out_hbm,
                  sidx, didx, u0, v0, u1, v1, res, sem0, sem1):
        wid = lax.axis_index("s") * 2 + lax.axis_index("c")
        row0 = wid * IDX_ROWS_TILE
        bufs = ((u0, v0, sem0), (u1, v1, sem1))

        pltpu.sync_copy(src_hbm.at[pl.ds(row0, IDX_ROWS_TILE)], sidx)
        pltpu.sync_copy(dst_hbm.at[pl.ds(row0, IDX_ROWS_TILE)], didx)

        def issue(c, buf):
            ub, vb, sem = buf
            for k in range(IDX_ROWS):
                r = c * IDX_ROWS + k
                pltpu.async_copy(h_hbm.at[sidx.at[r]],
                                 ub.at[pl.ds(k * 128, 128)], sem)
                pltpu.async_copy(h_hbm.at[didx.at[r]],
                                 vb.at[pl.ds(k * 128, 128)], sem)

        def drain(buf):
            ub, vb, sem = buf
            for k in range(IDX_ROWS):
                pltpu.make_async_copy(h_hbm.at[sidx.at[0]],
                                      ub.at[pl.ds(k * 128, 128)], sem).wait()
                pltpu.make_async_copy(h_hbm.at[didx.at[0]],
                                      vb.at[pl.ds(k * 128, 128)], sem).wait()

        def compute(c, buf):
            ub, vb, _ = buf

            @pl.loop(0, CHUNK // 64)
            def _block(b):
                lanes = lax.iota(jnp.int32, 16)
                rows = [b * 64 + t * 16 + lanes for t in range(4)]

                def jstep(jj, accs):
                    accs = list(accs)
                    for k in range(2):
                        jv = jnp.broadcast_to(jj * 2 + k, (16,))
                        jv = jv.astype(jnp.int32)
                        for t in range(4):
                            uu = plsc.load_gather(ub, [rows[t], jv])
                            vv = plsc.load_gather(vb, [rows[t], jv])
                            accs[t] = accs[t] + uu * vv
                    return tuple(accs)

                accs = lax.fori_loop(
                    0, D // 2, jstep,
                    tuple(jnp.zeros((16,), jnp.float32) for _ in range(4)))
                for t in range(4):
                    res[pl.ds(c * CHUNK + b * 64 + t * 16, 16)] = accs[t]

        issue(0, bufs[0])

        @pl.loop(0, N_CHUNKS, step=2)
        def _pair(c):
            with jax.named_scope("issue_a"):
                issue(c + 1, bufs[1])
            with jax.named_scope("drain_a"):
                drain(bufs[0])
            with jax.named_scope("compute_a"):
                compute(c, bufs[0])

            @pl.when(c + 2 < N_CHUNKS)
            def _():
                with jax.named_scope("issue_b"):
                    issue(c + 2, bufs[0])

            with jax.named_scope("drain_b"):
                drain(bufs[1])
            with jax.named_scope("compute_b"):
                compute(c + 1, bufs[1])

        pltpu.sync_copy(res, out_hbm.at[pl.ds(wid * EDGES_PER_TILE,
                                              EDGES_PER_TILE)])

    return sc_kernel(norm_h, src2d, dst2d)


def kernel(x, edge_index):
    norm_h = _normalize(x.astype(jnp.float32))
    ei = edge_index.astype(jnp.int32)
    pad = EDGES_PAD - N_EDGES
    src = jnp.concatenate([ei[0], jnp.zeros((pad,), jnp.int32)])
    dst = jnp.concatenate([ei[1], jnp.zeros((pad,), jnp.int32)])
    src2d = src.reshape(EDGES_PAD // 128, 128)
    dst2d = dst.reshape(EDGES_PAD // 128, 128)
    cos = _sc_cosine(norm_h, src2d, dst2d)
    return cos[:N_EDGES].reshape(N_EDGES, 1)


# speedup vs baseline: 2.0967x; 2.0967x over previous
import dataclasses
import functools

import jax
import jax.numpy as jnp
from jax import lax
from jax.experimental import pallas as pl
from jax.experimental.pallas import tpu as pltpu
from jax.experimental.pallas import tpu_sc as plsc

N_NODES = 10000
N_EDGES = 320000
D = 128
DW = D // 2

N_TILES = 32
EDGES_PAD = 327680
EDGES_PER_TILE = EDGES_PAD // N_TILES
CHUNK = 256
N_CHUNKS = EDGES_PER_TILE // CHUNK
IDX_ROWS = CHUNK // 128
IDX_ROWS_TILE = EDGES_PER_TILE // 128


def _normalize_body(x_ref, o_ref):
    xb = x_ref[...]
    ss = jnp.sum(xb * xb, axis=1, keepdims=True)
    norm = jnp.maximum(jnp.sqrt(ss), 1e-12)
    o_ref[...] = xb / norm


def _normalize(x):
    return pl.pallas_call(
        _normalize_body,
        out_shape=jax.ShapeDtypeStruct((N_NODES, D), jnp.float32),
        grid=(10,),
        in_specs=[pl.BlockSpec((N_NODES // 10, D), lambda i: (i, 0))],
        out_specs=pl.BlockSpec((N_NODES // 10, D), lambda i: (i, 0)),
    )(x)


def _sc_cosine(norm_h, src2d, dst2d):
    mesh = plsc.VectorSubcoreMesh(core_axis_name="c", subcore_axis_name="s")
    cp = pltpu.CompilerParams()
    if "needs_layout_passes" in pltpu.CompilerParams.__dataclass_fields__:
        cp = dataclasses.replace(cp, needs_layout_passes=False)
    cp = dataclasses.replace(cp, use_tc_tiling_on_sc=False)

    @functools.partial(
        pl.kernel,
        mesh=mesh,
        compiler_params=cp,
        out_type=jax.ShapeDtypeStruct((EDGES_PAD,), jnp.float32),
        scratch_types=[
            pltpu.VMEM((EDGES_PER_TILE,), jnp.int32),
            pltpu.VMEM((EDGES_PER_TILE,), jnp.int32),
            pltpu.VMEM((CHUNK, DW), jnp.int32),
            pltpu.VMEM((CHUNK, DW), jnp.int32),
            pltpu.VMEM((CHUNK, DW), jnp.int32),
            pltpu.VMEM((CHUNK, DW), jnp.int32),
            pltpu.VMEM((EDGES_PER_TILE,), jnp.float32),
            pltpu.SemaphoreType.DMA,
            pltpu.SemaphoreType.DMA,
        ],
    )
    def sc_kernel(h_hbm, src_hbm, dst_hbm, out_hbm,
                  sidx, didx, u0, v0, u1, v1, res, sem0, sem1):
        wid = lax.axis_index("s") * 2 + lax.axis_index("c")
        e0 = wid * EDGES_PER_TILE
        bufs = ((u0, v0, sem0), (u1, v1, sem1))

        pltpu.sync_copy(src_hbm.at[pl.ds(e0, EDGES_PER_TILE)], sidx)
        pltpu.sync_copy(dst_hbm.at[pl.ds(e0, EDGES_PER_TILE)], didx)

        def issue(c, buf):
            ub, vb, sem = buf
            pltpu.async_copy(h_hbm.at[sidx.at[pl.ds(c * CHUNK, CHUNK)]],
                             ub, sem)
            pltpu.async_copy(h_hbm.at[didx.at[pl.ds(c * CHUNK, CHUNK)]],
                             vb, sem)

        def drain(buf):
            ub, vb, sem = buf
            pltpu.make_async_copy(h_hbm.at[sidx.at[pl.ds(0, CHUNK)]],
                                  ub, sem).wait()
            pltpu.make_async_copy(h_hbm.at[didx.at[pl.ds(0, CHUNK)]],
                                  vb, sem).wait()

        def compute(c, buf):
            ub, vb, _ = buf

            @pl.loop(0, CHUNK // 64)
            def _block(b):
                lanes = lax.iota(jnp.int32, 16)
                rows = [b * 64 + t * 16 + lanes for t in range(4)]

                def jstep(jj, accs):
                    accs = list(accs)
                    j0 = jj * 2
                    jv0 = jnp.broadcast_to(j0, (16,)).astype(jnp.int32)
                    jv1 = jnp.broadcast_to(j0 + 1, (16,)).astype(jnp.int32)
                    for t in range(4):
                        uu0 = plsc.load_gather(ub, [rows[t], jv0])
                        vv0 = plsc.load_gather(vb, [rows[t], jv0])
                        uu1 = plsc.load_gather(ub, [rows[t], jv1])
                        vv1 = plsc.load_gather(vb, [rows[t], jv1])
                        p = (plsc.bitcast(uu0, jnp.bfloat16)
                             * plsc.bitcast(vv0, jnp.bfloat16)
                             + plsc.bitcast(uu1, jnp.bfloat16)
                             * plsc.bitcast(vv1, jnp.bfloat16))
                        pe, po = plsc.unpack(
                            p, format=plsc.PackFormat.INTERLEAVED)
                        accs[t] = accs[t] + pe + po
                    return tuple(accs)

                accs = lax.fori_loop(
                    0, DW // 2, jstep,
                    tuple(jnp.zeros((16,), jnp.float32) for _ in range(4)))
                for t in range(4):
                    res[pl.ds(c * CHUNK + b * 64 + t * 16, 16)] = accs[t]

        issue(0, bufs[0])

        @pl.loop(0, N_CHUNKS, step=2)
        def _pair(c):
            issue(c + 1, bufs[1])
            drain(bufs[0])
            compute(c, bufs[0])

            @pl.when(c + 2 < N_CHUNKS)
            def _():
                issue(c + 2, bufs[0])

            drain(bufs[1])
            compute(c + 1, bufs[1])

        pltpu.sync_copy(res, out_hbm.at[pl.ds(wid * EDGES_PER_TILE,
                                              EDGES_PER_TILE)])

    return sc_kernel(norm_h, src2d, dst2d)


def kernel(x, edge_index):
    norm_h = _normalize(x.astype(jnp.float32))
    norm_h = lax.bitcast_convert_type(
        norm_h.astype(jnp.bfloat16).reshape(N_NODES, DW, 2), jnp.int32)
    ei = edge_index.astype(jnp.int32)
    pad = EDGES_PAD - N_EDGES
    src = jnp.concatenate([ei[0], jnp.zeros((pad,), jnp.int32)])
    dst = jnp.concatenate([ei[1], jnp.zeros((pad,), jnp.int32)])
    cos = _sc_cosine(norm_h, src, dst)
    return cos[:N_EDGES].reshape(N_EDGES, 1)


# speedup vs baseline: 2.7006x; 1.2880x over previous
import dataclasses
import functools

import jax
import jax.numpy as jnp
from jax import lax
from jax.experimental import pallas as pl
from jax.experimental.pallas import tpu as pltpu
from jax.experimental.pallas import tpu_sc as plsc

N_NODES = 10000
N_EDGES = 320000
D = 128
DW = D // 2

N_TILES = 32
EDGES_PAD = 327680
EDGES_PER_TILE = EDGES_PAD // N_TILES
CHUNK = 256
N_CHUNKS = EDGES_PER_TILE // CHUNK
IDX_ROWS = CHUNK // 128
IDX_ROWS_TILE = EDGES_PER_TILE // 128


def _normalize_body(x_ref, o_ref):
    xb = x_ref[...]
    ss = jnp.sum(xb * xb, axis=1, keepdims=True)
    norm = jnp.maximum(jnp.sqrt(ss), 1e-12)
    o_ref[...] = xb / norm


def _normalize(x):
    return pl.pallas_call(
        _normalize_body,
        out_shape=jax.ShapeDtypeStruct((N_NODES, D), jnp.float32),
        grid=(10,),
        in_specs=[pl.BlockSpec((N_NODES // 10, D), lambda i: (i, 0))],
        out_specs=pl.BlockSpec((N_NODES // 10, D), lambda i: (i, 0)),
    )(x)


def _sc_cosine(norm_h, src2d, dst2d):
    mesh = plsc.VectorSubcoreMesh(core_axis_name="c", subcore_axis_name="s")
    cp = pltpu.CompilerParams()
    if "needs_layout_passes" in pltpu.CompilerParams.__dataclass_fields__:
        cp = dataclasses.replace(cp, needs_layout_passes=False)
    cp = dataclasses.replace(cp, use_tc_tiling_on_sc=False)

    @functools.partial(
        pl.kernel,
        mesh=mesh,
        compiler_params=cp,
        out_type=jax.ShapeDtypeStruct((EDGES_PAD,), jnp.float32),
        scratch_types=[
            pltpu.VMEM((EDGES_PER_TILE,), jnp.int32),
            pltpu.VMEM((EDGES_PER_TILE,), jnp.int32),
            pltpu.VMEM((CHUNK, DW), jnp.int32),
            pltpu.VMEM((CHUNK, DW), jnp.int32),
            pltpu.VMEM((CHUNK, DW), jnp.int32),
            pltpu.VMEM((CHUNK, DW), jnp.int32),
            pltpu.VMEM((EDGES_PER_TILE,), jnp.float32),
            pltpu.SemaphoreType.DMA,
            pltpu.SemaphoreType.DMA,
        ],
    )
    def sc_kernel(h_hbm, src_hbm, dst_hbm, out_hbm,
                  sidx, didx, u0, v0, u1, v1, res, sem0, sem1):
        wid = lax.axis_index("s") * 2 + lax.axis_index("c")
        e0 = wid * EDGES_PER_TILE
        bufs = ((u0, v0, sem0), (u1, v1, sem1))

        pltpu.sync_copy(src_hbm.at[pl.ds(e0, EDGES_PER_TILE)], sidx)
        pltpu.sync_copy(dst_hbm.at[pl.ds(e0, EDGES_PER_TILE)], didx)

        def issue(c, buf):
            ub, vb, sem = buf
            pltpu.async_copy(h_hbm.at[sidx.at[pl.ds(c * CHUNK, CHUNK)]],
                             ub, sem)
            pltpu.async_copy(h_hbm.at[didx.at[pl.ds(c * CHUNK, CHUNK)]],
                             vb, sem)

        def drain(buf):
            ub, vb, sem = buf
            pltpu.make_async_copy(h_hbm.at[sidx.at[pl.ds(0, CHUNK)]],
                                  ub, sem).wait()
            pltpu.make_async_copy(h_hbm.at[didx.at[pl.ds(0, CHUNK)]],
                                  vb, sem).wait()

        def compute(c, buf):
            ub, vb, _ = buf

            @pl.loop(0, CHUNK // 64)
            def _block(b):
                lanes = lax.iota(jnp.int32, 16)
                rows = [b * 64 + t * 16 + lanes for t in range(4)]

                def jstep(jj, accs):
                    accs = list(accs)
                    j0 = jj * 2
                    jv0 = (j0 + lanes) & (DW - 1)
                    jv1 = (j0 + 1 + lanes) & (DW - 1)
                    for t in range(4):
                        uu0 = plsc.load_gather(ub, [rows[t], jv0])
                        vv0 = plsc.load_gather(vb, [rows[t], jv0])
                        uu1 = plsc.load_gather(ub, [rows[t], jv1])
                        vv1 = plsc.load_gather(vb, [rows[t], jv1])
                        p = (plsc.bitcast(uu0, jnp.bfloat16)
                             * plsc.bitcast(vv0, jnp.bfloat16)
                             + plsc.bitcast(uu1, jnp.bfloat16)
                             * plsc.bitcast(vv1, jnp.bfloat16))
                        pe, po = plsc.unpack(
                            p, format=plsc.PackFormat.INTERLEAVED)
                        accs[t] = accs[t] + pe + po
                    return tuple(accs)

                accs = lax.fori_loop(
                    0, DW // 2, jstep,
                    tuple(jnp.zeros((16,), jnp.float32) for _ in range(4)))
                for t in range(4):
                    res[pl.ds(c * CHUNK + b * 64 + t * 16, 16)] = accs[t]

        issue(0, bufs[0])

        @pl.loop(0, N_CHUNKS, step=2)
        def _pair(c):
            issue(c + 1, bufs[1])
            drain(bufs[0])
            compute(c, bufs[0])

            @pl.when(c + 2 < N_CHUNKS)
            def _():
                issue(c + 2, bufs[0])

            drain(bufs[1])
            compute(c + 1, bufs[1])

        pltpu.sync_copy(res, out_hbm.at[pl.ds(wid * EDGES_PER_TILE,
                                              EDGES_PER_TILE)])

    return sc_kernel(norm_h, src2d, dst2d)


def kernel(x, edge_index):
    norm_h = _normalize(x.astype(jnp.float32))
    norm_h = lax.bitcast_convert_type(
        norm_h.astype(jnp.bfloat16).reshape(N_NODES, DW, 2), jnp.int32)
    ei = edge_index.astype(jnp.int32)
    pad = EDGES_PAD - N_EDGES
    src = jnp.concatenate([ei[0], jnp.zeros((pad,), jnp.int32)])
    dst = jnp.concatenate([ei[1], jnp.zeros((pad,), jnp.int32)])
    cos = _sc_cosine(norm_h, src, dst)
    return cos[:N_EDGES].reshape(N_EDGES, 1)


# speedup vs baseline: 2.7042x; 1.0013x over previous
import dataclasses
import functools

import jax
import jax.numpy as jnp
from jax import lax
from jax.experimental import pallas as pl
from jax.experimental.pallas import tpu as pltpu
from jax.experimental.pallas import tpu_sc as plsc

N_NODES = 10000
N_EDGES = 320000
D = 128
DW = D // 2

N_TILES = 32
EDGES_PAD = 327680
EDGES_PER_TILE = EDGES_PAD // N_TILES
CHUNK = 320
N_CHUNKS = EDGES_PER_TILE // CHUNK
IDX_ROWS = CHUNK // 128
IDX_ROWS_TILE = EDGES_PER_TILE // 128


def _normalize_body(x_ref, o_ref):
    xb = x_ref[...]
    ss = jnp.sum(xb * xb, axis=1, keepdims=True)
    norm = jnp.maximum(jnp.sqrt(ss), 1e-12)
    o_ref[...] = xb / norm


def _normalize(x):
    return pl.pallas_call(
        _normalize_body,
        out_shape=jax.ShapeDtypeStruct((N_NODES, D), jnp.float32),
        grid=(10,),
        in_specs=[pl.BlockSpec((N_NODES // 10, D), lambda i: (i, 0))],
        out_specs=pl.BlockSpec((N_NODES // 10, D), lambda i: (i, 0)),
    )(x)


def _sc_cosine(norm_h, src2d, dst2d):
    mesh = plsc.VectorSubcoreMesh(core_axis_name="c", subcore_axis_name="s")
    cp = pltpu.CompilerParams()
    if "needs_layout_passes" in pltpu.CompilerParams.__dataclass_fields__:
        cp = dataclasses.replace(cp, needs_layout_passes=False)
    cp = dataclasses.replace(cp, use_tc_tiling_on_sc=False)

    @functools.partial(
        pl.kernel,
        mesh=mesh,
        compiler_params=cp,
        out_type=jax.ShapeDtypeStruct((EDGES_PAD,), jnp.float32),
        scratch_types=[
            pltpu.VMEM((EDGES_PER_TILE,), jnp.int32),
            pltpu.VMEM((EDGES_PER_TILE,), jnp.int32),
            pltpu.VMEM((CHUNK, DW), jnp.int32),
            pltpu.VMEM((CHUNK, DW), jnp.int32),
            pltpu.VMEM((CHUNK, DW), jnp.int32),
            pltpu.VMEM((CHUNK, DW), jnp.int32),
            pltpu.VMEM((EDGES_PER_TILE,), jnp.float32),
            pltpu.SemaphoreType.DMA,
            pltpu.SemaphoreType.DMA,
        ],
    )
    def sc_kernel(h_hbm, src_hbm, dst_hbm, out_hbm,
                  sidx, didx, u0, v0, u1, v1, res, sem0, sem1):
        wid = lax.axis_index("s") * 2 + lax.axis_index("c")
        e0 = wid * EDGES_PER_TILE
        bufs = ((u0, v0, sem0), (u1, v1, sem1))

        pltpu.sync_copy(src_hbm.at[pl.ds(e0, EDGES_PER_TILE)], sidx)
        pltpu.sync_copy(dst_hbm.at[pl.ds(e0, EDGES_PER_TILE)], didx)

        def issue(c, buf):
            ub, vb, sem = buf
            pltpu.async_copy(h_hbm.at[sidx.at[pl.ds(c * CHUNK, CHUNK)]],
                             ub, sem)
            pltpu.async_copy(h_hbm.at[didx.at[pl.ds(c * CHUNK, CHUNK)]],
                             vb, sem)

        def drain(buf):
            ub, vb, sem = buf
            pltpu.make_async_copy(h_hbm.at[sidx.at[pl.ds(0, CHUNK)]],
                                  ub, sem).wait()
            pltpu.make_async_copy(h_hbm.at[didx.at[pl.ds(0, CHUNK)]],
                                  vb, sem).wait()

        def compute(c, buf):
            ub, vb, _ = buf

            @pl.loop(0, CHUNK // 64)
            def _block(b):
                lanes = lax.iota(jnp.int32, 16)
                rows = [b * 64 + t * 16 + lanes for t in range(4)]

                def jstep(jj, accs):
                    accs = list(accs)
                    j0 = jj * 2
                    jv0 = (j0 + lanes) & (DW - 1)
                    jv1 = (j0 + 1 + lanes) & (DW - 1)
                    for t in range(4):
                        uu0 = plsc.load_gather(ub, [rows[t], jv0])
                        vv0 = plsc.load_gather(vb, [rows[t], jv0])
                        uu1 = plsc.load_gather(ub, [rows[t], jv1])
                        vv1 = plsc.load_gather(vb, [rows[t], jv1])
                        p = (plsc.bitcast(uu0, jnp.bfloat16)
                             * plsc.bitcast(vv0, jnp.bfloat16)
                             + plsc.bitcast(uu1, jnp.bfloat16)
                             * plsc.bitcast(vv1, jnp.bfloat16))
                        pe, po = plsc.unpack(
                            p, format=plsc.PackFormat.INTERLEAVED)
                        accs[t] = accs[t] + pe + po
                    return tuple(accs)

                accs = lax.fori_loop(
                    0, DW // 2, jstep,
                    tuple(jnp.zeros((16,), jnp.float32) for _ in range(4)))
                for t in range(4):
                    res[pl.ds(c * CHUNK + b * 64 + t * 16, 16)] = accs[t]

        issue(0, bufs[0])

        @pl.loop(0, N_CHUNKS, step=2)
        def _pair(c):
            issue(c + 1, bufs[1])
            drain(bufs[0])
            compute(c, bufs[0])

            @pl.when(c + 2 < N_CHUNKS)
            def _():
                issue(c + 2, bufs[0])

            drain(bufs[1])
            compute(c + 1, bufs[1])

        pltpu.sync_copy(res, out_hbm.at[pl.ds(wid * EDGES_PER_TILE,
                                              EDGES_PER_TILE)])

    return sc_kernel(norm_h, src2d, dst2d)


def kernel(x, edge_index):
    norm_h = _normalize(x.astype(jnp.float32))
    norm_h = lax.bitcast_convert_type(
        norm_h.astype(jnp.bfloat16).reshape(N_NODES, DW, 2), jnp.int32)
    ei = edge_index.astype(jnp.int32)
    pad = EDGES_PAD - N_EDGES
    src = jnp.concatenate([ei[0], jnp.zeros((pad,), jnp.int32)])
    dst = jnp.concatenate([ei[1], jnp.zeros((pad,), jnp.int32)])
    cos = _sc_cosine(norm_h, src, dst)
    return cos[:N_EDGES].reshape(N_EDGES, 1)
